# baseline (speedup 1.0000x reference)
import functools

import jax
import jax.numpy as jnp
from jax import lax
from jax.experimental import pallas as pl
from jax.experimental.pallas import tpu as pltpu
from jax.experimental.pallas import tpu_sc as plsc

N = 10000
E = 320000
DIM = 128
AUG = 144

NC = 2
NS = 16
NW = NC * NS
EPW = E // NW
CHUNK = 50
NCHUNK = EPW // CHUNK
ROWS_PT = N // NS
ZROWS = 25


def _sc_body_with_acc(xaug_hbm, src_hbm, dst_hbm, out_hbm,
                      acc, src_v, dst_v, rows_a, rows_b,
                      sem_ga, sem_gb, sem_sa, sem_sb):
    c = lax.axis_index("c")
    s = lax.axis_index("s")
    wid = c * NS + s

    zero16 = jnp.zeros((16,), jnp.float32)
    for r in range(ZROWS):
        for q in range(AUG // 16):
            rows_a[r, pl.ds(q * 16, 16)] = zero16

    pltpu.sync_copy(src_hbm.at[pl.ds(wid * NCHUNK, NCHUNK)], src_v)
    pltpu.sync_copy(dst_hbm.at[pl.ds(wid * NCHUNK, NCHUNK)], dst_v)

    def _zero_step(i, carry):
        pltpu.sync_copy(rows_a.at[pl.ds(0, ZROWS)],
                        acc.at[pl.ds(s * ROWS_PT + i * ZROWS, ZROWS)])
        return carry

    lax.fori_loop(0, ROWS_PT // ZROWS, _zero_step, 0)
    plsc.subcore_barrier()

    dummy = xaug_hbm.at[pl.ds(0, CHUNK)]
    pltpu.async_copy(xaug_hbm.at[src_v.at[0]], rows_a, sem_ga)
    pltpu.async_copy(xaug_hbm.at[src_v.at[1]], rows_b, sem_gb)

    def _pair_step(i, carry):
        pltpu.make_async_copy(dummy, rows_a, sem_ga).wait()
        pltpu.async_copy(rows_a, acc.at[dst_v.at[2 * i]], sem_sa, add=True)
        pltpu.make_async_copy(dummy, rows_b, sem_gb).wait()
        pltpu.async_copy(rows_b, acc.at[dst_v.at[2 * i + 1]], sem_sb, add=True)
        pltpu.make_async_copy(dummy, rows_a, sem_sa).wait()
        ga = jnp.minimum(2 * i + 2, NCHUNK - 1)
        pltpu.async_copy(xaug_hbm.at[src_v.at[ga]], rows_a, sem_ga)
        pltpu.make_async_copy(dummy, rows_b, sem_sb).wait()
        gb = jnp.minimum(2 * i + 3, NCHUNK - 1)
        pltpu.async_copy(xaug_hbm.at[src_v.at[gb]], rows_b, sem_gb)
        return carry

    lax.fori_loop(0, NCHUNK // 2, _pair_step, 0)
    pltpu.make_async_copy(dummy, rows_a, sem_ga).wait()
    pltpu.make_async_copy(dummy, rows_b, sem_gb).wait()
    plsc.subcore_barrier()

    pltpu.sync_copy(acc.at[pl.ds(s * ROWS_PT, ROWS_PT)],
                    out_hbm.at[c, pl.ds(s * ROWS_PT, ROWS_PT)])


_sc_aggregate = pl.kernel(
    _sc_body_with_acc,
    out_type=jax.ShapeDtypeStruct((NC, N, AUG), jnp.float32),
    mesh=plsc.VectorSubcoreMesh(core_axis_name="c", subcore_axis_name="s"),
    compiler_params=pltpu.CompilerParams(use_tc_tiling_on_sc=False),
    scratch_types=[
        pltpu.VMEM_SHARED((N, AUG), jnp.float32),
        pltpu.VMEM((NCHUNK, CHUNK), jnp.int32),
        pltpu.VMEM((NCHUNK, CHUNK), jnp.int32),
        pltpu.VMEM((CHUNK, AUG), jnp.float32),
        pltpu.VMEM((CHUNK, AUG), jnp.float32),
        pltpu.SemaphoreType.DMA,
        pltpu.SemaphoreType.DMA,
        pltpu.SemaphoreType.DMA,
        pltpu.SemaphoreType.DMA,
    ],
)


RB = 400


def _combine_body(p_ref, x_ref, wl_ref, bl_ref, wr_ref, o_ref):
    acc = p_ref[0] + p_ref[1]
    cnt = jnp.maximum(acc[:, DIM:DIM + 1], 1.0)
    mean = acc[:, :DIM] / cnt
    h = lax.dot_general(mean, wl_ref[...], (((1,), (1,)), ((), ())),
                        precision=lax.Precision.HIGHEST,
                        preferred_element_type=jnp.float32)
    h = h + lax.dot_general(x_ref[...], wr_ref[...], (((1,), (1,)), ((), ())),
                            precision=lax.Precision.HIGHEST,
                            preferred_element_type=jnp.float32)
    h = h + bl_ref[...]
    o_ref[...] = jnp.maximum(h, 0.0)


def _tc_combine(partials, x, W_l, b_l2, W_r):
    return pl.pallas_call(
        _combine_body,
        grid=(N // RB,),
        in_specs=[
            pl.BlockSpec((NC, RB, AUG), lambda i: (0, i, 0)),
            pl.BlockSpec((RB, DIM), lambda i: (i, 0)),
            pl.BlockSpec((DIM, DIM), lambda i: (0, 0)),
            pl.BlockSpec((1, DIM), lambda i: (0, 0)),
            pl.BlockSpec((DIM, DIM), lambda i: (0, 0)),
        ],
        out_specs=pl.BlockSpec((RB, DIM), lambda i: (i, 0)),
        out_shape=jax.ShapeDtypeStruct((N, DIM), jnp.float32),
    )(partials, x, W_l, b_l2, W_r)


def kernel(x, edge_index, W_l, b_l, W_r):
    ei = edge_index.astype(jnp.int32)
    src = ei[0].reshape(NW * NCHUNK, CHUNK)
    dst = ei[1].reshape(NW * NCHUNK, CHUNK)
    xaug = jnp.concatenate(
        [x, jnp.ones((N, 1), jnp.float32), jnp.zeros((N, AUG - DIM - 1), jnp.float32)],
        axis=1)
    partials = _sc_aggregate(xaug, src, dst)
    return _tc_combine(partials, x, W_l, b_l.reshape(1, DIM), W_r)


# speedup vs baseline: 1.0427x; 1.0427x over previous
import functools

import jax
import jax.numpy as jnp
from jax import lax
from jax.experimental import pallas as pl
from jax.experimental.pallas import tpu as pltpu
from jax.experimental.pallas import tpu_sc as plsc

N = 10000
E = 320000
DIM = 128
AUG = 144

NC = 2
NS = 16
NW = NC * NS
EPW = E // NW
CHUNK = 40
NCHUNK = EPW // CHUNK
ROWS_PT = N // NS
ZROWS = 25


def _sc_body_with_acc(xaug_hbm, src_hbm, dst_hbm, out_hbm,
                      acc, src_v, dst_v, rows_a, rows_b,
                      sem_ga, sem_gb, sem_sa, sem_sb):
    c = lax.axis_index("c")
    s = lax.axis_index("s")
    wid = c * NS + s

    zero16 = jnp.zeros((16,), jnp.float32)
    for r in range(ZROWS):
        for q in range(AUG // 16):
            rows_a[r, pl.ds(q * 16, 16)] = zero16

    pltpu.sync_copy(src_hbm.at[pl.ds(wid * NCHUNK, NCHUNK)], src_v)
    pltpu.sync_copy(dst_hbm.at[pl.ds(wid * NCHUNK, NCHUNK)], dst_v)

    def _zero_step(i, carry):
        pltpu.sync_copy(rows_a.at[pl.ds(0, ZROWS)],
                        acc.at[pl.ds(s * ROWS_PT + i * ZROWS, ZROWS)])
        return carry

    lax.fori_loop(0, ROWS_PT // ZROWS, _zero_step, 0)
    plsc.subcore_barrier()

    dummy = xaug_hbm.at[pl.ds(0, CHUNK)]
    pltpu.async_copy(xaug_hbm.at[src_v.at[0]], rows_a, sem_ga)
    pltpu.async_copy(xaug_hbm.at[src_v.at[1]], rows_b, sem_gb)

    def _pair_step(i, carry):
        pltpu.make_async_copy(dummy, rows_a, sem_ga).wait()
        pltpu.async_copy(rows_a, acc.at[dst_v.at[2 * i]], sem_sa, add=True)
        pltpu.make_async_copy(dummy, rows_b, sem_gb).wait()
        pltpu.async_copy(rows_b, acc.at[dst_v.at[2 * i + 1]], sem_sb, add=True)
        pltpu.make_async_copy(dummy, rows_a, sem_sa).wait()
        ga = jnp.minimum(2 * i + 2, NCHUNK - 1)
        pltpu.async_copy(xaug_hbm.at[src_v.at[ga]], rows_a, sem_ga)
        pltpu.make_async_copy(dummy, rows_b, sem_sb).wait()
        gb = jnp.minimum(2 * i + 3, NCHUNK - 1)
        pltpu.async_copy(xaug_hbm.at[src_v.at[gb]], rows_b, sem_gb)
        return carry

    lax.fori_loop(0, NCHUNK // 2, _pair_step, 0)
    pltpu.make_async_copy(dummy, rows_a, sem_ga).wait()
    pltpu.make_async_copy(dummy, rows_b, sem_gb).wait()
    plsc.subcore_barrier()

    pltpu.sync_copy(acc.at[pl.ds(s * ROWS_PT, ROWS_PT)],
                    out_hbm.at[c, pl.ds(s * ROWS_PT, ROWS_PT)])


_sc_aggregate = pl.kernel(
    _sc_body_with_acc,
    out_type=jax.ShapeDtypeStruct((NC, N, AUG), jnp.float32),
    mesh=plsc.VectorSubcoreMesh(core_axis_name="c", subcore_axis_name="s"),
    compiler_params=pltpu.CompilerParams(use_tc_tiling_on_sc=False),
    scratch_types=[
        pltpu.VMEM_SHARED((N, AUG), jnp.float32),
        pltpu.VMEM((NCHUNK, CHUNK), jnp.int32),
        pltpu.VMEM((NCHUNK, CHUNK), jnp.int32),
        pltpu.VMEM((CHUNK, AUG), jnp.float32),
        pltpu.VMEM((CHUNK, AUG), jnp.float32),
        pltpu.SemaphoreType.DMA,
        pltpu.SemaphoreType.DMA,
        pltpu.SemaphoreType.DMA,
        pltpu.SemaphoreType.DMA,
    ],
)


RB = 400


def _combine_body(p_ref, x_ref, wl_ref, bl_ref, wr_ref, o_ref):
    acc = p_ref[0] + p_ref[1]
    cnt = jnp.maximum(acc[:, DIM:DIM + 1], 1.0)
    mean = acc[:, :DIM] / cnt
    h = lax.dot_general(mean, wl_ref[...], (((1,), (1,)), ((), ())),
                        precision=lax.Precision.HIGHEST,
                        preferred_element_type=jnp.float32)
    h = h + lax.dot_general(x_ref[...], wr_ref[...], (((1,), (1,)), ((), ())),
                            precision=lax.Precision.HIGHEST,
                            preferred_element_type=jnp.float32)
    h = h + bl_ref[...]
    o_ref[...] = jnp.maximum(h, 0.0)


def _tc_combine(partials, x, W_l, b_l2, W_r):
    return pl.pallas_call(
        _combine_body,
        grid=(N // RB,),
        in_specs=[
            pl.BlockSpec((NC, RB, AUG), lambda i: (0, i, 0)),
            pl.BlockSpec((RB, DIM), lambda i: (i, 0)),
            pl.BlockSpec((DIM, DIM), lambda i: (0, 0)),
            pl.BlockSpec((1, DIM), lambda i: (0, 0)),
            pl.BlockSpec((DIM, DIM), lambda i: (0, 0)),
        ],
        out_specs=pl.BlockSpec((RB, DIM), lambda i: (i, 0)),
        out_shape=jax.ShapeDtypeStruct((N, DIM), jnp.float32),
    )(partials, x, W_l, b_l2, W_r)


def kernel(x, edge_index, W_l, b_l, W_r):
    ei = edge_index.astype(jnp.int32)
    src = ei[0].reshape(NW * NCHUNK, CHUNK)
    dst = ei[1].reshape(NW * NCHUNK, CHUNK)
    xaug = jnp.concatenate(
        [x, jnp.ones((N, 1), jnp.float32), jnp.zeros((N, AUG - DIM - 1), jnp.float32)],
        axis=1)
    partials = _sc_aggregate(xaug, src, dst)
    return _tc_combine(partials, x, W_l, b_l.reshape(1, DIM), W_r)


# speedup vs baseline: 1.4079x; 1.3502x over previous
import functools

import jax
import jax.numpy as jnp
from jax import lax
from jax.experimental import pallas as pl
from jax.experimental.pallas import tpu as pltpu
from jax.experimental.pallas import tpu_sc as plsc

N = 10000
E = 320000
DIM = 128

NC = 2
NS = 16
NW = NC * NS
EPW = E // NW
CHUNK = 80
NCHUNK = EPW // CHUNK
ROWS_PT = N // NS
ZROWS = 25
QSTEPS = CHUNK // 16


def _sc_body_with_acc(x_hbm, src_hbm, dst_hbm, out_hbm, hist_hbm,
                      acc, src_v, dst_v, rows_a, rows_b, hist,
                      sem_ga, sem_gb, sem_sa, sem_sb):
    c = lax.axis_index("c")
    s = lax.axis_index("s")
    wid = c * NS + s

    zero16 = jnp.zeros((16,), jnp.float32)
    for r in range(ZROWS):
        for q in range(DIM // 16):
            rows_a[r, pl.ds(q * 16, 16)] = zero16

    pltpu.sync_copy(src_hbm.at[pl.ds(wid * NCHUNK, NCHUNK)], src_v)
    pltpu.sync_copy(dst_hbm.at[pl.ds(wid * NCHUNK, NCHUNK)], dst_v)

    def _zero_step(i, carry):
        pltpu.sync_copy(rows_a.at[pl.ds(0, ZROWS)],
                        acc.at[pl.ds(s * ROWS_PT + i * ZROWS, ZROWS)])
        return carry

    lax.fori_loop(0, ROWS_PT // ZROWS, _zero_step, 0)

    def _zero_hist(i, carry):
        hist[pl.ds(i * 16, 16)] = zero16
        return carry

    lax.fori_loop(0, N // 16, _zero_hist, 0)
    plsc.subcore_barrier()

    ones16 = jnp.ones((16,), jnp.float32)

    def _hist_chunk(j):
        for q in range(QSTEPS):
            idx = dst_v[j, pl.ds(q * 16, 16)]
            plsc.addupdate_scatter(hist, [idx], ones16)

    dummy = x_hbm.at[pl.ds(0, CHUNK)]
    pltpu.async_copy(x_hbm.at[src_v.at[0]], rows_a, sem_ga)
    pltpu.async_copy(x_hbm.at[src_v.at[1]], rows_b, sem_gb)

    def _pair_step(i, carry):
        pltpu.make_async_copy(dummy, rows_a, sem_ga).wait()
        pltpu.async_copy(rows_a, acc.at[dst_v.at[2 * i]], sem_sa, add=True)
        _hist_chunk(2 * i)
        pltpu.make_async_copy(dummy, rows_b, sem_gb).wait()
        pltpu.async_copy(rows_b, acc.at[dst_v.at[2 * i + 1]], sem_sb, add=True)
        pltpu.make_async_copy(dummy, rows_a, sem_sa).wait()
        ga = jnp.minimum(2 * i + 2, NCHUNK - 1)
        pltpu.async_copy(x_hbm.at[src_v.at[ga]], rows_a, sem_ga)
        _hist_chunk(2 * i + 1)
        pltpu.make_async_copy(dummy, rows_b, sem_sb).wait()
        gb = jnp.minimum(2 * i + 3, NCHUNK - 1)
        pltpu.async_copy(x_hbm.at[src_v.at[gb]], rows_b, sem_gb)
        return carry

    lax.fori_loop(0, NCHUNK // 2, _pair_step, 0)
    pltpu.make_async_copy(dummy, rows_a, sem_ga).wait()
    pltpu.async_copy(rows_a, acc.at[dst_v.at[NCHUNK - 1]], sem_sa, add=True)
    _hist_chunk(NCHUNK - 1)
    pltpu.make_async_copy(dummy, rows_b, sem_gb).wait()
    pltpu.make_async_copy(dummy, rows_a, sem_sa).wait()
    plsc.subcore_barrier()

    pltpu.sync_copy(acc.at[pl.ds(s * ROWS_PT, ROWS_PT)],
                    out_hbm.at[c, pl.ds(s * ROWS_PT, ROWS_PT)])
    pltpu.sync_copy(hist, hist_hbm.at[c, s])


_sc_aggregate = pl.kernel(
    _sc_body_with_acc,
    out_type=[jax.ShapeDtypeStruct((NC, N, DIM), jnp.float32),
              jax.ShapeDtypeStruct((NC, NS, N), jnp.float32)],
    mesh=plsc.VectorSubcoreMesh(core_axis_name="c", subcore_axis_name="s"),
    compiler_params=pltpu.CompilerParams(use_tc_tiling_on_sc=False,
                                         needs_layout_passes=False),
    scratch_types=[
        pltpu.VMEM_SHARED((N, DIM), jnp.float32),
        pltpu.VMEM((NCHUNK, CHUNK), jnp.int32),
        pltpu.VMEM((NCHUNK, CHUNK), jnp.int32),
        pltpu.VMEM((CHUNK, DIM), jnp.float32),
        pltpu.VMEM((CHUNK, DIM), jnp.float32),
        pltpu.VMEM((N,), jnp.float32),
        pltpu.SemaphoreType.DMA,
        pltpu.SemaphoreType.DMA,
        pltpu.SemaphoreType.DMA,
        pltpu.SemaphoreType.DMA,
    ],
)


RB = 400


def _combine_body(p_ref, h_ref, x_ref, wl_ref, bl_ref, wr_ref, o_ref):
    acc = p_ref[0] + p_ref[1]
    cnt = jnp.sum(h_ref[...], axis=1)
    cnt = jnp.maximum(cnt, 1.0)[:, None]
    mean = acc / cnt
    h = lax.dot_general(mean, wl_ref[...], (((1,), (1,)), ((), ())),
                        precision=lax.Precision.HIGHEST,
                        preferred_element_type=jnp.float32)
    h = h + lax.dot_general(x_ref[...], wr_ref[...], (((1,), (1,)), ((), ())),
                            precision=lax.Precision.HIGHEST,
                            preferred_element_type=jnp.float32)
    h = h + bl_ref[...]
    o_ref[...] = jnp.maximum(h, 0.0)


def _tc_combine(partials, hists, x, W_l, b_l2, W_r):
    return pl.pallas_call(
        _combine_body,
        grid=(N // RB,),
        in_specs=[
            pl.BlockSpec((NC, RB, DIM), lambda i: (0, i, 0)),
            pl.BlockSpec((RB, NW), lambda i: (i, 0)),
            pl.BlockSpec((RB, DIM), lambda i: (i, 0)),
            pl.BlockSpec((DIM, DIM), lambda i: (0, 0)),
            pl.BlockSpec((1, DIM), lambda i: (0, 0)),
            pl.BlockSpec((DIM, DIM), lambda i: (0, 0)),
        ],
        out_specs=pl.BlockSpec((RB, DIM), lambda i: (i, 0)),
        out_shape=jax.ShapeDtypeStruct((N, DIM), jnp.float32),
    )(partials, hists, x, W_l, b_l2, W_r)


def kernel(x, edge_index, W_l, b_l, W_r):
    ei = edge_index.astype(jnp.int32)
    src = ei[0].reshape(NW * NCHUNK, CHUNK)
    dst = ei[1].reshape(NW * NCHUNK, CHUNK)
    partials, hists = _sc_aggregate(x, src, dst)
    hs = hists.reshape(NW, N).T
    return _tc_combine(partials, hs, x, W_l, b_l.reshape(1, DIM), W_r)


# speedup vs baseline: 1.4132x; 1.0038x over previous
import functools

import jax
import jax.numpy as jnp
from jax import lax
from jax.experimental import pallas as pl
from jax.experimental.pallas import tpu as pltpu
from jax.experimental.pallas import tpu_sc as plsc

N = 10000
E = 320000
DIM = 128

NC = 2
NS = 16
NW = NC * NS
EPW = E // NW
CHUNK = 80
NCHUNK = EPW // CHUNK
ROWS_PT = N // NS
ZROWS = 25
QSTEPS = CHUNK // 16


def _sc_body_with_acc(x_hbm, src_hbm, dst_hbm, out_hbm, hist_hbm,
                      acc, src_v, dst_v, rows_a, rows_b, hist,
                      sem_ga, sem_gb, sem_sa, sem_sb):
    c = lax.axis_index("c")
    s = lax.axis_index("s")
    wid = c * NS + s

    zero16 = jnp.zeros((16,), jnp.float32)
    for r in range(ZROWS):
        for q in range(DIM // 16):
            rows_a[r, pl.ds(q * 16, 16)] = zero16

    pltpu.sync_copy(src_hbm.at[pl.ds(wid * NCHUNK, NCHUNK)], src_v)
    pltpu.sync_copy(dst_hbm.at[pl.ds(wid * NCHUNK, NCHUNK)], dst_v)

    def _zero_step(i, carry):
        pltpu.sync_copy(rows_a.at[pl.ds(0, ZROWS)],
                        acc.at[pl.ds(s * ROWS_PT + i * ZROWS, ZROWS)])
        return carry

    lax.fori_loop(0, ROWS_PT // ZROWS, _zero_step, 0)

    def _zero_hist(i, carry):
        hist[pl.ds(i * 16, 16)] = zero16
        return carry

    lax.fori_loop(0, N // 16, _zero_hist, 0)
    plsc.subcore_barrier()

    ones16 = jnp.ones((16,), jnp.float32)

    def _hist_chunk(j):
        for q in range(QSTEPS):
            idx = dst_v[j, pl.ds(q * 16, 16)]
            plsc.addupdate_scatter(hist, [idx], ones16)

    dummy = x_hbm.at[pl.ds(0, CHUNK)]
    pltpu.async_copy(x_hbm.at[src_v.at[0]], rows_a, sem_ga)
    pltpu.async_copy(x_hbm.at[src_v.at[1]], rows_b, sem_gb)

    def _pair_step(i, carry):
        pltpu.make_async_copy(dummy, rows_a, sem_ga).wait()
        pltpu.async_copy(rows_a, acc.at[dst_v.at[2 * i]], sem_sa, add=True)
        _hist_chunk(2 * i)
        pltpu.make_async_copy(dummy, rows_b, sem_gb).wait()
        pltpu.async_copy(rows_b, acc.at[dst_v.at[2 * i + 1]], sem_sb, add=True)
        pltpu.make_async_copy(dummy, rows_a, sem_sa).wait()
        ga = jnp.minimum(2 * i + 2, NCHUNK - 1)
        pltpu.async_copy(x_hbm.at[src_v.at[ga]], rows_a, sem_ga)
        _hist_chunk(2 * i + 1)
        pltpu.make_async_copy(dummy, rows_b, sem_sb).wait()
        gb = jnp.minimum(2 * i + 3, NCHUNK - 1)
        pltpu.async_copy(x_hbm.at[src_v.at[gb]], rows_b, sem_gb)
        return carry

    lax.fori_loop(0, NCHUNK // 2, _pair_step, 0)
    pltpu.make_async_copy(dummy, rows_a, sem_ga).wait()
    pltpu.async_copy(rows_a, acc.at[dst_v.at[NCHUNK - 1]], sem_sa, add=True)
    _hist_chunk(NCHUNK - 1)
    pltpu.make_async_copy(dummy, rows_b, sem_gb).wait()
    pltpu.make_async_copy(dummy, rows_a, sem_sa).wait()
    plsc.subcore_barrier()

    pltpu.sync_copy(acc.at[pl.ds(s * ROWS_PT, ROWS_PT)],
                    out_hbm.at[c, pl.ds(s * ROWS_PT, ROWS_PT)])
    pltpu.sync_copy(hist, hist_hbm.at[c, s])


_sc_aggregate = pl.kernel(
    _sc_body_with_acc,
    out_type=[jax.ShapeDtypeStruct((NC, N, DIM), jnp.float32),
              jax.ShapeDtypeStruct((NC, NS, N), jnp.float32)],
    mesh=plsc.VectorSubcoreMesh(core_axis_name="c", subcore_axis_name="s"),
    compiler_params=pltpu.CompilerParams(use_tc_tiling_on_sc=False,
                                         needs_layout_passes=False),
    scratch_types=[
        pltpu.VMEM_SHARED((N, DIM), jnp.float32),
        pltpu.VMEM((NCHUNK, CHUNK), jnp.int32),
        pltpu.VMEM((NCHUNK, CHUNK), jnp.int32),
        pltpu.VMEM((CHUNK, DIM), jnp.float32),
        pltpu.VMEM((CHUNK, DIM), jnp.float32),
        pltpu.VMEM((N,), jnp.float32),
        pltpu.SemaphoreType.DMA,
        pltpu.SemaphoreType.DMA,
        pltpu.SemaphoreType.DMA,
        pltpu.SemaphoreType.DMA,
    ],
)


RB = 400


def _xr_body(x_ref, wr_ref, bl_ref, o_ref):
    h = lax.dot_general(x_ref[...], wr_ref[...], (((1,), (1,)), ((), ())),
                        precision=lax.Precision.HIGHEST,
                        preferred_element_type=jnp.float32)
    o_ref[...] = h + bl_ref[...]


def _tc_xr(x, W_r, b_l2):
    return pl.pallas_call(
        _xr_body,
        grid=(N // RB,),
        in_specs=[
            pl.BlockSpec((RB, DIM), lambda i: (i, 0)),
            pl.BlockSpec((DIM, DIM), lambda i: (0, 0)),
            pl.BlockSpec((1, DIM), lambda i: (0, 0)),
        ],
        out_specs=pl.BlockSpec((RB, DIM), lambda i: (i, 0)),
        out_shape=jax.ShapeDtypeStruct((N, DIM), jnp.float32),
    )(x, W_r, b_l2)


def _combine_body(p_ref, h_ref, xr_ref, wl_ref, o_ref):
    acc = p_ref[0] + p_ref[1]
    cnt = jnp.sum(h_ref[...], axis=1)
    cnt = jnp.maximum(cnt, 1.0)[:, None]
    mean = acc / cnt
    h = lax.dot_general(mean, wl_ref[...], (((1,), (1,)), ((), ())),
                        precision=lax.Precision.HIGHEST,
                        preferred_element_type=jnp.float32)
    h = h + xr_ref[...]
    o_ref[...] = jnp.maximum(h, 0.0)


def _tc_combine(partials, hists, xr, W_l):
    return pl.pallas_call(
        _combine_body,
        grid=(N // RB,),
        in_specs=[
            pl.BlockSpec((NC, RB, DIM), lambda i: (0, i, 0)),
            pl.BlockSpec((RB, NW), lambda i: (i, 0)),
            pl.BlockSpec((RB, DIM), lambda i: (i, 0)),
            pl.BlockSpec((DIM, DIM), lambda i: (0, 0)),
        ],
        out_specs=pl.BlockSpec((RB, DIM), lambda i: (i, 0)),
        out_shape=jax.ShapeDtypeStruct((N, DIM), jnp.float32),
    )(partials, hists, xr, W_l)


def kernel(x, edge_index, W_l, b_l, W_r):
    ei = edge_index.astype(jnp.int32)
    src = ei[0].reshape(NW * NCHUNK, CHUNK)
    dst = ei[1].reshape(NW * NCHUNK, CHUNK)
    xr = _tc_xr(x, W_r, b_l.reshape(1, DIM))
    partials, hists = _sc_aggregate(x, src, dst)
    hs = hists.reshape(NW, N).T
    return _tc_combine(partials, hs, xr, W_l)


# speedup vs baseline: 1.4884x; 1.0532x over previous
import functools

import jax
import jax.numpy as jnp
from jax import lax
from jax.experimental import pallas as pl
from jax.experimental.pallas import tpu as pltpu
from jax.experimental.pallas import tpu_sc as plsc

N = 10000
E = 320000
DIM = 128

NC = 2
NS = 16
NW = NC * NS
EPW = E // NW
CHUNK = 80
NCHUNK = EPW // CHUNK
ROWS_PT = N // NS
ZROWS = 25
QSTEPS = CHUNK // 16


def _sc_body_with_acc(x_hbm, src_hbm, dst_hbm, out_hbm, hist_hbm,
                      acc, src_v, dst_v, rows_a, rows_b, hist,
                      sem_ga, sem_gb, sem_sa, sem_sb):
    c = lax.axis_index("c")
    s = lax.axis_index("s")
    wid = c * NS + s

    zero16 = jnp.zeros((16,), jnp.float32)
    for r in range(ZROWS):
        for q in range(DIM // 16):
            rows_a[r, pl.ds(q * 16, 16)] = zero16

    pltpu.sync_copy(src_hbm.at[pl.ds(wid * NCHUNK, NCHUNK)], src_v)
    pltpu.sync_copy(dst_hbm.at[pl.ds(wid * NCHUNK, NCHUNK)], dst_v)

    def _zero_step(i, carry):
        pltpu.sync_copy(rows_a.at[pl.ds(0, ZROWS)],
                        acc.at[pl.ds(s * ROWS_PT + i * ZROWS, ZROWS)])
        return carry

    lax.fori_loop(0, ROWS_PT // ZROWS, _zero_step, 0)

    def _zero_hist(i, carry):
        hist[pl.ds(i * 16, 16)] = zero16
        return carry

    lax.fori_loop(0, N // 16, _zero_hist, 0)
    plsc.subcore_barrier()

    ones16 = jnp.ones((16,), jnp.float32)

    def _hist_chunk(j):
        for q in range(QSTEPS):
            idx = dst_v[j, pl.ds(q * 16, 16)]
            plsc.addupdate_scatter(hist, [idx], ones16)

    dummy = x_hbm.at[pl.ds(0, CHUNK)]
    pltpu.async_copy(x_hbm.at[src_v.at[0]], rows_a, sem_ga)
    pltpu.async_copy(x_hbm.at[src_v.at[1]], rows_b, sem_gb)

    def _pair_step(i, carry):
        pltpu.make_async_copy(dummy, rows_a, sem_ga).wait()
        pltpu.async_copy(rows_a, acc.at[dst_v.at[2 * i]], sem_sa, add=True)
        _hist_chunk(2 * i)
        pltpu.make_async_copy(dummy, rows_b, sem_gb).wait()
        pltpu.async_copy(rows_b, acc.at[dst_v.at[2 * i + 1]], sem_sb, add=True)
        pltpu.make_async_copy(dummy, rows_a, sem_sa).wait()
        ga = jnp.minimum(2 * i + 2, NCHUNK - 1)
        pltpu.async_copy(x_hbm.at[src_v.at[ga]], rows_a, sem_ga)
        _hist_chunk(2 * i + 1)
        pltpu.make_async_copy(dummy, rows_b, sem_sb).wait()
        gb = jnp.minimum(2 * i + 3, NCHUNK - 1)
        pltpu.async_copy(x_hbm.at[src_v.at[gb]], rows_b, sem_gb)
        return carry

    lax.fori_loop(0, NCHUNK // 2, _pair_step, 0)
    pltpu.make_async_copy(dummy, rows_a, sem_ga).wait()
    pltpu.async_copy(rows_a, acc.at[dst_v.at[NCHUNK - 1]], sem_sa, add=True)
    _hist_chunk(NCHUNK - 1)
    pltpu.make_async_copy(dummy, rows_b, sem_gb).wait()
    pltpu.make_async_copy(dummy, rows_a, sem_sa).wait()
    plsc.subcore_barrier()

    pltpu.sync_copy(acc.at[pl.ds(s * ROWS_PT, ROWS_PT)],
                    out_hbm.at[c, pl.ds(s * ROWS_PT, ROWS_PT)])
    pltpu.sync_copy(hist, hist_hbm.at[c, s])


_sc_aggregate = pl.kernel(
    _sc_body_with_acc,
    out_type=[jax.ShapeDtypeStruct((NC, N, DIM), jnp.float32),
              jax.ShapeDtypeStruct((NC, NS, N), jnp.float32)],
    mesh=plsc.VectorSubcoreMesh(core_axis_name="c", subcore_axis_name="s"),
    compiler_params=pltpu.CompilerParams(use_tc_tiling_on_sc=False,
                                         needs_layout_passes=False),
    scratch_types=[
        pltpu.VMEM_SHARED((N, DIM), jnp.float32),
        pltpu.VMEM((NCHUNK, CHUNK), jnp.int32),
        pltpu.VMEM((NCHUNK, CHUNK), jnp.int32),
        pltpu.VMEM((CHUNK, DIM), jnp.float32),
        pltpu.VMEM((CHUNK, DIM), jnp.float32),
        pltpu.VMEM((N,), jnp.float32),
        pltpu.SemaphoreType.DMA,
        pltpu.SemaphoreType.DMA,
        pltpu.SemaphoreType.DMA,
        pltpu.SemaphoreType.DMA,
    ],
)


RB = 2000


def _xr_body(x_ref, wr_ref, bl_ref, o_ref):
    h = lax.dot_general(x_ref[...], wr_ref[...], (((1,), (1,)), ((), ())),
                        precision=lax.Precision.HIGHEST,
                        preferred_element_type=jnp.float32)
    o_ref[...] = h + bl_ref[...]


def _tc_xr(x, W_r, b_l2):
    return pl.pallas_call(
        _xr_body,
        grid=(N // RB,),
        in_specs=[
            pl.BlockSpec((RB, DIM), lambda i: (i, 0)),
            pl.BlockSpec((DIM, DIM), lambda i: (0, 0)),
            pl.BlockSpec((1, DIM), lambda i: (0, 0)),
        ],
        out_specs=pl.BlockSpec((RB, DIM), lambda i: (i, 0)),
        out_shape=jax.ShapeDtypeStruct((N, DIM), jnp.float32),
    )(x, W_r, b_l2)


def _combine_body(p_ref, h_ref, xr_ref, wl_ref, o_ref):
    acc = p_ref[0] + p_ref[1]
    cnt = jnp.sum(h_ref[...], axis=1)
    cnt = jnp.maximum(cnt, 1.0)[:, None]
    mean = acc / cnt
    h = lax.dot_general(mean, wl_ref[...], (((1,), (1,)), ((), ())),
                        precision=lax.Precision.HIGHEST,
                        preferred_element_type=jnp.float32)
    h = h + xr_ref[...]
    o_ref[...] = jnp.maximum(h, 0.0)


def _tc_combine(partials, hists, xr, W_l):
    return pl.pallas_call(
        _combine_body,
        grid=(N // RB,),
        in_specs=[
            pl.BlockSpec((NC, RB, DIM), lambda i: (0, i, 0)),
            pl.BlockSpec((RB, NW), lambda i: (i, 0)),
            pl.BlockSpec((RB, DIM), lambda i: (i, 0)),
            pl.BlockSpec((DIM, DIM), lambda i: (0, 0)),
        ],
        out_specs=pl.BlockSpec((RB, DIM), lambda i: (i, 0)),
        out_shape=jax.ShapeDtypeStruct((N, DIM), jnp.float32),
    )(partials, hists, xr, W_l)


def kernel(x, edge_index, W_l, b_l, W_r):
    ei = edge_index.astype(jnp.int32)
    src = ei[0].reshape(NW * NCHUNK, CHUNK)
    dst = ei[1].reshape(NW * NCHUNK, CHUNK)
    xr = _tc_xr(x, W_r, b_l.reshape(1, DIM))
    partials, hists = _sc_aggregate(x, src, dst)
    hs = hists.reshape(NW, N).T
    return _tc_combine(partials, hs, xr, W_l)


# speedup vs baseline: 1.5025x; 1.0094x over previous
import functools

import jax
import jax.numpy as jnp
from jax import lax
from jax.experimental import pallas as pl
from jax.experimental.pallas import tpu as pltpu
from jax.experimental.pallas import tpu_sc as plsc

N = 10000
E = 320000
DIM = 128

NC = 2
NS = 16
NW = NC * NS
EPW = E // NW
CHUNK = 80
NCHUNK = EPW // CHUNK
ROWS_PT = N // NS
ZROWS = 25
QSTEPS = CHUNK // 16


def _sc_body_with_acc(x_hbm, src_hbm, dst_hbm, out_hbm, hist_hbm,
                      acc, src_v, dst_v, rows_a, rows_b, hist,
                      sem_ga, sem_gb, sem_sa, sem_sb):
    c = lax.axis_index("c")
    s = lax.axis_index("s")
    wid = c * NS + s

    zero16 = jnp.zeros((16,), jnp.float32)
    for r in range(ZROWS):
        for q in range(DIM // 16):
            rows_a[r, pl.ds(q * 16, 16)] = zero16

    pltpu.sync_copy(src_hbm.at[pl.ds(wid * NCHUNK, NCHUNK)], src_v)
    pltpu.sync_copy(dst_hbm.at[pl.ds(wid * NCHUNK, NCHUNK)], dst_v)

    def _zero_step(i, carry):
        pltpu.sync_copy(rows_a.at[pl.ds(0, ZROWS)],
                        acc.at[pl.ds(s * ROWS_PT + i * ZROWS, ZROWS)])
        return carry

    lax.fori_loop(0, ROWS_PT // ZROWS, _zero_step, 0)

    def _zero_hist(i, carry):
        hist[pl.ds(i * 16, 16)] = zero16
        return carry

    lax.fori_loop(0, N // 16, _zero_hist, 0)
    plsc.subcore_barrier()

    ones16 = jnp.ones((16,), jnp.float32)

    def _hist_chunk(j):
        for q in range(QSTEPS):
            idx = dst_v[j, pl.ds(q * 16, 16)]
            plsc.addupdate_scatter(hist, [idx], ones16)

    dummy = x_hbm.at[pl.ds(0, CHUNK)]
    pltpu.async_copy(x_hbm.at[src_v.at[0]], rows_a, sem_ga)
    pltpu.async_copy(x_hbm.at[src_v.at[1]], rows_b, sem_gb)

    def _pair_step(i, carry):
        pltpu.make_async_copy(dummy, rows_a, sem_ga).wait()
        pltpu.async_copy(rows_a, acc.at[dst_v.at[2 * i]], sem_sa, add=True)
        _hist_chunk(2 * i)
        pltpu.make_async_copy(dummy, rows_b, sem_gb).wait()
        pltpu.async_copy(rows_b, acc.at[dst_v.at[2 * i + 1]], sem_sb, add=True)
        pltpu.make_async_copy(dummy, rows_a, sem_sa).wait()
        ga = jnp.minimum(2 * i + 2, NCHUNK - 1)
        pltpu.async_copy(x_hbm.at[src_v.at[ga]], rows_a, sem_ga)
        _hist_chunk(2 * i + 1)
        pltpu.make_async_copy(dummy, rows_b, sem_sb).wait()
        gb = jnp.minimum(2 * i + 3, NCHUNK - 1)
        pltpu.async_copy(x_hbm.at[src_v.at[gb]], rows_b, sem_gb)
        return carry

    lax.fori_loop(0, NCHUNK // 2, _pair_step, 0)
    pltpu.make_async_copy(dummy, rows_a, sem_ga).wait()
    pltpu.async_copy(rows_a, acc.at[dst_v.at[NCHUNK - 1]], sem_sa, add=True)
    _hist_chunk(NCHUNK - 1)
    pltpu.make_async_copy(dummy, rows_b, sem_gb).wait()
    pltpu.make_async_copy(dummy, rows_a, sem_sa).wait()
    plsc.subcore_barrier()

    pltpu.sync_copy(acc.at[pl.ds(s * ROWS_PT, ROWS_PT)],
                    out_hbm.at[c, pl.ds(s * ROWS_PT, ROWS_PT)])
    pltpu.sync_copy(hist, hist_hbm.at[c, s])


_sc_aggregate = pl.kernel(
    _sc_body_with_acc,
    out_type=[jax.ShapeDtypeStruct((NC, N, DIM), jnp.float32),
              jax.ShapeDtypeStruct((NC, NS, N), jnp.float32)],
    mesh=plsc.VectorSubcoreMesh(core_axis_name="c", subcore_axis_name="s"),
    compiler_params=pltpu.CompilerParams(use_tc_tiling_on_sc=False,
                                         needs_layout_passes=False),
    scratch_types=[
        pltpu.VMEM_SHARED((N, DIM), jnp.float32),
        pltpu.VMEM((NCHUNK, CHUNK), jnp.int32),
        pltpu.VMEM((NCHUNK, CHUNK), jnp.int32),
        pltpu.VMEM((CHUNK, DIM), jnp.float32),
        pltpu.VMEM((CHUNK, DIM), jnp.float32),
        pltpu.VMEM((N,), jnp.float32),
        pltpu.SemaphoreType.DMA,
        pltpu.SemaphoreType.DMA,
        pltpu.SemaphoreType.DMA,
        pltpu.SemaphoreType.DMA,
    ],
)


RB = 2000


def _xr_body(x_ref, wr_ref, bl_ref, o_ref):
    h = lax.dot_general(x_ref[...], wr_ref[...], (((1,), (1,)), ((), ())),
                        precision=lax.Precision.DEFAULT,
                        preferred_element_type=jnp.float32)
    o_ref[...] = h + bl_ref[...]


def _tc_xr(x, W_r, b_l2):
    return pl.pallas_call(
        _xr_body,
        grid=(N // RB,),
        in_specs=[
            pl.BlockSpec((RB, DIM), lambda i: (i, 0)),
            pl.BlockSpec((DIM, DIM), lambda i: (0, 0)),
            pl.BlockSpec((1, DIM), lambda i: (0, 0)),
        ],
        out_specs=pl.BlockSpec((RB, DIM), lambda i: (i, 0)),
        out_shape=jax.ShapeDtypeStruct((N, DIM), jnp.float32),
    )(x, W_r, b_l2)


def _combine_body(p_ref, h_ref, xr_ref, wl_ref, o_ref):
    acc = p_ref[0] + p_ref[1]
    cnt = jnp.sum(h_ref[...], axis=1)
    cnt = jnp.maximum(cnt, 1.0)[:, None]
    mean = acc / cnt
    h = lax.dot_general(mean, wl_ref[...], (((1,), (1,)), ((), ())),
                        precision=lax.Precision.DEFAULT,
                        preferred_element_type=jnp.float32)
    h = h + xr_ref[...]
    o_ref[...] = jnp.maximum(h, 0.0)


def _tc_combine(partials, hists, xr, W_l):
    return pl.pallas_call(
        _combine_body,
        grid=(N // RB,),
        in_specs=[
            pl.BlockSpec((NC, RB, DIM), lambda i: (0, i, 0)),
            pl.BlockSpec((RB, NW), lambda i: (i, 0)),
            pl.BlockSpec((RB, DIM), lambda i: (i, 0)),
            pl.BlockSpec((DIM, DIM), lambda i: (0, 0)),
        ],
        out_specs=pl.BlockSpec((RB, DIM), lambda i: (i, 0)),
        out_shape=jax.ShapeDtypeStruct((N, DIM), jnp.float32),
    )(partials, hists, xr, W_l)


def kernel(x, edge_index, W_l, b_l, W_r):
    ei = edge_index.astype(jnp.int32)
    src = ei[0].reshape(NW * NCHUNK, CHUNK)
    dst = ei[1].reshape(NW * NCHUNK, CHUNK)
    xr = _tc_xr(x, W_r, b_l.reshape(1, DIM))
    partials, hists = _sc_aggregate(x, src, dst)
    hs = hists.reshape(NW, N).T
    return _tc_combine(partials, hs, xr, W_l)
